# baseline (speedup 1.0000x reference)
import functools

import jax
import jax.numpy as jnp
from jax.experimental import pallas as pl
from jax.experimental.pallas import tpu as pltpu

_BF = jnp.bfloat16
_F32 = jnp.float32


def _band_matrix(w, wp):
    eyes = jnp.stack([jnp.eye(wp, k=1 - kw, dtype=_F32) for kw in range(3)])
    cout, cin = w.shape[0], w.shape[1]
    b = jnp.einsum("ochk,kuv->hcuov", w.astype(_F32), eyes)
    return b.reshape(3 * cin * wp, cout * wp).astype(_BF)


def _conv_kernel(x_ref, b1_ref, r1_ref, b2_ref, r2_ref, b3_ref, r3_ref,
                 o_ref, s1, s2, *, m, r, rp, hp, wp):
    def layer(src, b_ref, bias_ref, cout, dout):
        xk = jnp.concatenate(
            [src[0:m, :], src[1:m + 1, :], src[2:m + 2, :]], axis=1)
        t = jnp.dot(xk, b_ref[...], preferred_element_type=_F32)
        t = jnp.maximum(t + bias_ref[...], 0.0)
        rows = jax.lax.broadcasted_iota(jnp.int32, (m, 1), 0)
        q = rows - dout
        qm = jnp.remainder(q, hp)
        rowok = (q >= 0) & (q < r) & (qm >= 1) & (qm < hp - 1)
        cols = jax.lax.broadcasted_iota(jnp.int32, (1, cout * wp), 1)
        wm = jnp.remainder(cols, wp)
        colok = (wm >= 1) & (wm < wp - 1)
        return jnp.where(rowok & colok, t, 0.0).astype(_BF)

    c1 = r1_ref.shape[1] // wp
    c2 = r2_ref.shape[1] // wp
    c3 = r3_ref.shape[1] // wp

    a1 = layer(x_ref, b1_ref, r1_ref, c1, 2)
    s1[0:m, :] = a1
    s1[m:rp, :] = jnp.zeros((rp - m, s1.shape[1]), _BF)
    a2 = layer(s1, b2_ref, r2_ref, c2, 1)
    s2[0:m, :] = a2
    s2[m:rp, :] = jnp.zeros((rp - m, s2.shape[1]), _BF)
    a3 = layer(s2, b3_ref, r3_ref, c3, 0)
    o_ref[...] = a3[0:r, :]


def _fc_kernel(x_ref, w_ref, b_ref, o_ref):
    o_ref[...] = (jnp.dot(x_ref[...], w_ref[...],
                          preferred_element_type=_F32) + b_ref[...])


def kernel(conv1_w, conv1_b, conv2_w, conv2_b, conv3_w, conv3_b,
           fc_w, fc_b, x):
    n, c0, h, w = x.shape
    c1, c2, c3 = conv1_w.shape[0], conv2_w.shape[0], conv3_w.shape[0]
    ncls = fc_w.shape[0]
    hp, wp = h + 2, w + 2

    nb = 32
    while n % nb:
        nb //= 2
    g = n // nb
    r = nb * hp
    m = r + 8
    rp = r + 16

    b1 = _band_matrix(conv1_w, wp)
    b2 = _band_matrix(conv2_w, wp)
    b3 = _band_matrix(conv3_w, wp)
    r1 = jnp.repeat(conv1_b.astype(_F32), wp).reshape(1, c1 * wp)
    r2 = jnp.repeat(conv2_b.astype(_F32), wp).reshape(1, c2 * wp)
    r3 = jnp.repeat(conv3_b.astype(_F32), wp).reshape(1, c3 * wp)

    xp = jnp.pad(x.astype(_BF), ((0, 0), (0, 0), (1, 1), (1, 1)))
    xt = xp.transpose(0, 2, 1, 3).reshape(g, r, c0 * wp)
    xt = jnp.pad(xt, ((0, 0), (3, rp - r - 3), (0, 0)))

    conv_body = functools.partial(_conv_kernel, m=m, r=r, rp=rp, hp=hp, wp=wp)
    a3 = pl.pallas_call(
        conv_body,
        out_shape=jax.ShapeDtypeStruct((g, r, c3 * wp), _BF),
        grid=(g,),
        in_specs=[
            pl.BlockSpec((None, rp, c0 * wp), lambda i: (i, 0, 0)),
            pl.BlockSpec(b1.shape, lambda i: (0, 0)),
            pl.BlockSpec(r1.shape, lambda i: (0, 0)),
            pl.BlockSpec(b2.shape, lambda i: (0, 0)),
            pl.BlockSpec(r2.shape, lambda i: (0, 0)),
            pl.BlockSpec(b3.shape, lambda i: (0, 0)),
            pl.BlockSpec(r3.shape, lambda i: (0, 0)),
        ],
        out_specs=pl.BlockSpec((None, r, c3 * wp), lambda i: (i, 0, 0)),
        scratch_shapes=[
            pltpu.VMEM((rp, c1 * wp), _BF),
            pltpu.VMEM((rp, c2 * wp), _BF),
        ],
        compiler_params=pltpu.CompilerParams(
            dimension_semantics=("arbitrary",)),
    )(xt, b1, r1, b2, r2, b3, r3)

    a3f = a3.reshape(n, hp * c3 * wp)
    fw = jnp.pad(fc_w.reshape(ncls, c3, h, w).astype(_F32),
                 ((0, 0), (0, 0), (1, 1), (1, 1)))
    fw = fw.transpose(2, 1, 3, 0).reshape(hp * c3 * wp, ncls).astype(_BF)
    fb = fc_b.astype(_F32).reshape(1, ncls)

    out = pl.pallas_call(
        _fc_kernel,
        out_shape=jax.ShapeDtypeStruct((n, ncls), _F32),
        grid=(1,),
        in_specs=[
            pl.BlockSpec(a3f.shape, lambda i: (0, 0)),
            pl.BlockSpec(fw.shape, lambda i: (0, 0)),
            pl.BlockSpec(fb.shape, lambda i: (0, 0)),
        ],
        out_specs=pl.BlockSpec((n, ncls), lambda i: (0, 0)),
        compiler_params=pltpu.CompilerParams(
            dimension_semantics=("arbitrary",)),
    )(a3f, fw, fb)
    return out


# speedup vs baseline: 15.4134x; 15.4134x over previous
import functools

import jax
import jax.numpy as jnp
from jax.experimental import pallas as pl
from jax.experimental.pallas import tpu as pltpu

_BF = jnp.bfloat16
_F32 = jnp.float32


def _band_matrix(w, wp):
    eyes = jnp.stack([jnp.eye(wp, k=1 - kw, dtype=_F32) for kw in range(3)])
    cout, cin = w.shape[0], w.shape[1]
    b = jnp.einsum("ochk,kuv->hcuov", w.astype(_F32), eyes)
    return b.reshape(3 * cin * wp, cout * wp).astype(_BF)


def _conv_kernel(x_ref, b1_ref, r1_ref, b2_ref, r2_ref, b3_ref, r3_ref,
                 o_ref, s0, s1, s2, *, m, r, rp, hp, wp):
    nb, c0, h, w = x_ref.shape

    s0[...] = jnp.zeros((rp, c0 * wp), _BF)
    for n in range(nb):
        base = 3 + n * hp + 1
        for c in range(c0):
            s0[base:base + h, c * wp + 1:c * wp + 1 + w] = (
                x_ref[n, c].astype(_BF))

    def layer(src, b_ref, bias_ref, cout, dout):
        xk = jnp.concatenate(
            [src[0:m, :], src[1:m + 1, :], src[2:m + 2, :]], axis=1)
        t = jnp.dot(xk, b_ref[...], preferred_element_type=_F32)
        t = jnp.maximum(t + bias_ref[...], 0.0)
        rows = jax.lax.broadcasted_iota(jnp.int32, (m, 1), 0)
        q = rows - dout
        qm = jnp.remainder(q, hp)
        rowok = (q >= 0) & (q < r) & (qm >= 1) & (qm < hp - 1)
        cols = jax.lax.broadcasted_iota(jnp.int32, (1, cout * wp), 1)
        wm = jnp.remainder(cols, wp)
        colok = (wm >= 1) & (wm < wp - 1)
        return jnp.where(rowok & colok, t, 0.0).astype(_BF)

    c1 = r1_ref.shape[1] // wp
    c2 = r2_ref.shape[1] // wp
    c3 = r3_ref.shape[1] // wp

    a1 = layer(s0, b1_ref, r1_ref, c1, 2)
    s1[0:m, :] = a1
    s1[m:rp, :] = jnp.zeros((rp - m, s1.shape[1]), _BF)
    a2 = layer(s1, b2_ref, r2_ref, c2, 1)
    s2[0:m, :] = a2
    s2[m:rp, :] = jnp.zeros((rp - m, s2.shape[1]), _BF)
    a3 = layer(s2, b3_ref, r3_ref, c3, 0)
    o_ref[...] = a3[0:r, :]


def _fc_kernel(x_ref, w_ref, b_ref, o_ref):
    o_ref[...] = (jnp.dot(x_ref[...], w_ref[...],
                          preferred_element_type=_F32) + b_ref[...])


def kernel(conv1_w, conv1_b, conv2_w, conv2_b, conv3_w, conv3_b,
           fc_w, fc_b, x):
    n, c0, h, w = x.shape
    c1, c2, c3 = conv1_w.shape[0], conv2_w.shape[0], conv3_w.shape[0]
    ncls = fc_w.shape[0]
    hp, wp = h + 2, w + 2

    nb = 32
    while n % nb:
        nb //= 2
    g = n // nb
    r = nb * hp
    m = r + 8
    rp = r + 16

    b1 = _band_matrix(conv1_w, wp)
    b2 = _band_matrix(conv2_w, wp)
    b3 = _band_matrix(conv3_w, wp)
    r1 = jnp.repeat(conv1_b.astype(_F32), wp).reshape(1, c1 * wp)
    r2 = jnp.repeat(conv2_b.astype(_F32), wp).reshape(1, c2 * wp)
    r3 = jnp.repeat(conv3_b.astype(_F32), wp).reshape(1, c3 * wp)

    conv_body = functools.partial(_conv_kernel, m=m, r=r, rp=rp, hp=hp, wp=wp)
    a3 = pl.pallas_call(
        conv_body,
        out_shape=jax.ShapeDtypeStruct((g, r, c3 * wp), _BF),
        grid=(g,),
        in_specs=[
            pl.BlockSpec((nb, c0, h, w), lambda i: (i, 0, 0, 0)),
            pl.BlockSpec(b1.shape, lambda i: (0, 0)),
            pl.BlockSpec(r1.shape, lambda i: (0, 0)),
            pl.BlockSpec(b2.shape, lambda i: (0, 0)),
            pl.BlockSpec(r2.shape, lambda i: (0, 0)),
            pl.BlockSpec(b3.shape, lambda i: (0, 0)),
            pl.BlockSpec(r3.shape, lambda i: (0, 0)),
        ],
        out_specs=pl.BlockSpec((None, r, c3 * wp), lambda i: (i, 0, 0)),
        scratch_shapes=[
            pltpu.VMEM((rp, c0 * wp), _BF),
            pltpu.VMEM((rp, c1 * wp), _BF),
            pltpu.VMEM((rp, c2 * wp), _BF),
        ],
        compiler_params=pltpu.CompilerParams(
            dimension_semantics=("arbitrary",)),
    )(x, b1, r1, b2, r2, b3, r3)

    a3f = a3.reshape(n, hp * c3 * wp)
    fw = jnp.pad(fc_w.reshape(ncls, c3, h, w).astype(_F32),
                 ((0, 0), (0, 0), (1, 1), (1, 1)))
    fw = fw.transpose(2, 1, 3, 0).reshape(hp * c3 * wp, ncls).astype(_BF)
    fb = fc_b.astype(_F32).reshape(1, ncls)

    out = pl.pallas_call(
        _fc_kernel,
        out_shape=jax.ShapeDtypeStruct((n, ncls), _F32),
        grid=(1,),
        in_specs=[
            pl.BlockSpec(a3f.shape, lambda i: (0, 0)),
            pl.BlockSpec(fw.shape, lambda i: (0, 0)),
            pl.BlockSpec(fb.shape, lambda i: (0, 0)),
        ],
        out_specs=pl.BlockSpec((n, ncls), lambda i: (0, 0)),
        compiler_params=pltpu.CompilerParams(
            dimension_semantics=("arbitrary",)),
    )(a3f, fw, fb)
    return out


# speedup vs baseline: 17.8042x; 1.1551x over previous
import functools

import jax
import jax.numpy as jnp
from jax.experimental import pallas as pl
from jax.experimental.pallas import tpu as pltpu

_BF = jnp.bfloat16
_F32 = jnp.float32


def _band_matrix(w, wp):
    eyes = jnp.stack([jnp.eye(wp, k=1 - kw, dtype=_F32) for kw in range(3)])
    colmask = ((jnp.arange(wp) >= 1) & (jnp.arange(wp) < wp - 1))
    eyes = eyes * colmask[None, None, :].astype(_F32)
    cout, cin = w.shape[0], w.shape[1]
    b = jnp.einsum("ochk,kuv->hcuov", w.astype(_F32), eyes)
    return b.reshape(3 * cin * wp, cout * wp).astype(_BF)


def _conv_kernel(x_ref, b1_ref, r1_ref, b2_ref, r2_ref, b3_ref, r3_ref,
                 fw_ref, fb_ref, o_ref, s0, s1, s2, *, m, r, rp, hp, wp):
    nb, c0, h, w = x_ref.shape

    s0[...] = jnp.zeros((rp, c0 * wp), _BF)
    for n in range(nb):
        base = 3 + n * hp + 1
        for c in range(c0):
            s0[base:base + h, c * wp + 1:c * wp + 1 + w] = x_ref[n, c]

    def layer(src, b_ref, bias_ref, cout, dout):
        xk = jnp.concatenate(
            [src[0:m, :], src[1:m + 1, :], src[2:m + 2, :]], axis=1)
        t = jnp.dot(xk, b_ref[...], preferred_element_type=_F32)
        t = jnp.maximum(t + bias_ref[...], 0.0)
        rows = jax.lax.broadcasted_iota(jnp.int32, (m, 1), 0)
        qm = jnp.remainder(rows - dout, hp)
        rowok = (qm >= 1) & (qm < hp - 1)
        return jnp.where(rowok, t, 0.0).astype(_BF)

    c1 = r1_ref.shape[1] // wp
    c2 = r2_ref.shape[1] // wp
    c3 = r3_ref.shape[1] // wp

    a1 = layer(s0, b1_ref, r1_ref, c1, 2)
    s1[0:m, :] = a1
    s1[m:rp, :] = jnp.zeros((rp - m, s1.shape[1]), _BF)
    a2 = layer(s1, b2_ref, r2_ref, c2, 1)
    s2[0:m, :] = a2
    s2[m:rp, :] = jnp.zeros((rp - m, s2.shape[1]), _BF)
    a3 = layer(s2, b3_ref, r3_ref, c3, 0)

    ncls = fb_ref.shape[1]
    nbi = o_ref.shape[0]
    p = jnp.dot(a3[0:r], fw_ref[...], preferred_element_type=_F32)
    prow = jnp.remainder(jax.lax.broadcasted_iota(jnp.int32, (r, 1), 0), hp)
    pcol = jax.lax.broadcasted_iota(jnp.int32, (1, hp * ncls), 1) // ncls
    pm = jnp.where(prow == pcol, p, 0.0).astype(_BF)
    sel = (jax.lax.broadcasted_iota(jnp.int32, (nbi, r), 1) // hp ==
           jax.lax.broadcasted_iota(jnp.int32, (nbi, r), 0))
    q1 = jnp.dot(sel.astype(_BF), pm, preferred_element_type=_F32)
    e = (jnp.remainder(jax.lax.broadcasted_iota(jnp.int32, (hp * ncls, ncls),
                                                0), ncls) ==
         jax.lax.broadcasted_iota(jnp.int32, (hp * ncls, ncls), 1))
    o_ref[...] = (jnp.dot(q1.astype(_BF), e.astype(_BF),
                          preferred_element_type=_F32) + fb_ref[...])


def kernel(conv1_w, conv1_b, conv2_w, conv2_b, conv3_w, conv3_b,
           fc_w, fc_b, x):
    n, c0, h, w = x.shape
    c1, c2, c3 = conv1_w.shape[0], conv2_w.shape[0], conv3_w.shape[0]
    ncls = fc_w.shape[0]
    hp, wp = h + 2, w + 2

    nb = 32
    while n % nb:
        nb //= 2
    g = n // nb
    r = nb * hp
    m = r + 8
    rp = r + 16

    b1 = _band_matrix(conv1_w, wp)
    b2 = _band_matrix(conv2_w, wp)
    b3 = _band_matrix(conv3_w, wp)
    bcol = ((jnp.arange(wp) >= 1) & (jnp.arange(wp) < wp - 1)).astype(_F32)
    r1 = (conv1_b.astype(_F32)[:, None] * bcol).reshape(1, c1 * wp)
    r2 = (conv2_b.astype(_F32)[:, None] * bcol).reshape(1, c2 * wp)
    r3 = (conv3_b.astype(_F32)[:, None] * bcol).reshape(1, c3 * wp)

    fw2 = jnp.pad(fc_w.reshape(ncls, c3, h, w).astype(_F32),
                  ((0, 0), (0, 0), (1, 1), (1, 1)))
    fw2 = fw2.transpose(1, 3, 2, 0).reshape(c3 * wp, hp * ncls).astype(_BF)
    fb = fc_b.astype(_F32).reshape(1, ncls)

    conv_body = functools.partial(_conv_kernel, m=m, r=r, rp=rp, hp=hp, wp=wp)
    out = pl.pallas_call(
        conv_body,
        out_shape=jax.ShapeDtypeStruct((g, nb, ncls), _F32),
        grid=(g,),
        in_specs=[
            pl.BlockSpec((nb, c0, h, w), lambda i: (i, 0, 0, 0)),
            pl.BlockSpec(b1.shape, lambda i: (0, 0)),
            pl.BlockSpec(r1.shape, lambda i: (0, 0)),
            pl.BlockSpec(b2.shape, lambda i: (0, 0)),
            pl.BlockSpec(r2.shape, lambda i: (0, 0)),
            pl.BlockSpec(b3.shape, lambda i: (0, 0)),
            pl.BlockSpec(r3.shape, lambda i: (0, 0)),
            pl.BlockSpec(fw2.shape, lambda i: (0, 0)),
            pl.BlockSpec(fb.shape, lambda i: (0, 0)),
        ],
        out_specs=pl.BlockSpec((None, nb, ncls), lambda i: (i, 0, 0)),
        scratch_shapes=[
            pltpu.VMEM((rp, c0 * wp), _BF),
            pltpu.VMEM((rp, c1 * wp), _BF),
            pltpu.VMEM((rp, c2 * wp), _BF),
        ],
        compiler_params=pltpu.CompilerParams(
            dimension_semantics=("arbitrary",)),
    )(x.astype(_BF), b1, r1, b2, r2, b3, r3, fw2, fb)
    return out.reshape(n, ncls)
